# Initial kernel scaffold; baseline (speedup 1.0000x reference)
import jax
import jax.numpy as jnp
from jax.experimental import pallas as pl


def kernel(distance_matrixA, weight, bias, index_ls):
    raise NotImplementedError("write your pallas kernel here")


# speedup vs baseline: 7.8823x; 7.8823x over previous
import jax
import jax.numpy as jnp
from jax import lax
from jax.experimental import pallas as pl
from jax.experimental.pallas import tpu as pltpu
from jax.experimental.pallas import tpu_sc as plsc

_K = 16


def _full_i(x):
    return jnp.full((_K,), x, dtype=jnp.int32)


def _full_f(x):
    return jnp.full((_K,), x, dtype=jnp.float32)


def _sc_body(a_hbm, w_hbm, b_hbm, i_hbm, out_hbm, a_v, w_v, b_v, sel_v):
    c = lax.axis_index("c")
    s = lax.axis_index("s")

    @pl.when(jnp.logical_and(c == 0, s == 0))
    def _():
        pltpu.sync_copy(a_hbm, a_v)
        pltpu.sync_copy(w_hbm, w_v)
        pltpu.sync_copy(b_hbm, b_v)
        pltpu.sync_copy(i_hbm, sel_v)

        iota = lax.iota(jnp.int32, _K)
        idx0 = plsc.load_gather(sel_v, [_full_i(0)])
        mask0 = iota == idx0
        bias = b_v[...]
        inf = _full_f(jnp.inf)

        def step(L, mask):
            Lv = _full_i(L)

            def inner(j, score):
                jv = _full_i(j)
                sj = plsc.load_gather(sel_v, [jv])
                wi = jnp.maximum(Lv - 1 - jv, _full_i(0))
                wj = plsc.load_gather(w_v, [wi])
                col = plsc.load_gather(a_v, [iota, sj])
                contrib = jnp.where(jv < Lv, col * wj, _full_f(0.0))
                return score + contrib

            score = lax.fori_loop(0, _K, inner, _full_f(0.0))
            score = score + bias
            score = jnp.maximum(score, _full_f(0.0))
            score = jnp.where(score == _full_f(0.0), inf, score)
            score = jnp.where(mask, inf, score)
            m = jnp.min(score)
            cand = jnp.where(
                jnp.logical_and(jnp.logical_not(mask), score == _full_f(m)),
                iota,
                _full_i(_K),
            )
            nv = _full_i(jnp.min(cand))
            sel_v[...] = jnp.where(iota == Lv, nv, sel_v[...])
            return jnp.logical_or(mask, iota == nv)

        lax.fori_loop(1, _K, step, mask0)
        pltpu.sync_copy(sel_v, out_hbm)


def kernel(distance_matrixA, weight, bias, index_ls):
    w_pad = jnp.zeros((_K,), jnp.float32).at[: weight.shape[1]].set(weight[0])
    b_pad = jnp.full((_K,), bias[0], dtype=jnp.float32)
    i_pad = (
        jnp.zeros((_K,), jnp.int32)
        .at[: index_ls.shape[0]]
        .set(index_ls.astype(jnp.int32))
    )
    mesh = plsc.VectorSubcoreMesh(core_axis_name="c", subcore_axis_name="s")
    f = pl.kernel(
        _sc_body,
        out_type=jax.ShapeDtypeStruct((_K,), jnp.int32),
        mesh=mesh,
        compiler_params=pltpu.CompilerParams(needs_layout_passes=False),
        scratch_types=[
            pltpu.VMEM((_K, _K), jnp.float32),
            pltpu.VMEM((_K,), jnp.float32),
            pltpu.VMEM((_K,), jnp.float32),
            pltpu.VMEM((_K,), jnp.int32),
        ],
    )
    out_index = f(distance_matrixA, w_pad, b_pad, i_pad)
    return (out_index, weight)


# speedup vs baseline: 9.7136x; 1.2323x over previous
import jax
import jax.numpy as jnp
from jax import lax
from jax.experimental import pallas as pl
from jax.experimental.pallas import tpu as pltpu
from jax.experimental.pallas import tpu_sc as plsc

_K = 16


def _full_i(x):
    return jnp.full((_K,), x, dtype=jnp.int32)


def _full_f(x):
    return jnp.full((_K,), x, dtype=jnp.float32)


def _bcast(vec, lane):
    return jnp.take_along_axis(
        vec, _full_i(lane), axis=0, mode="promise_in_bounds"
    )


def _sc_body(p_hbm, out_hbm, p_v, o_v):
    pltpu.sync_copy(p_hbm, p_v)

    iota = lax.iota(jnp.int32, _K)
    w_row = p_v[_K, :]
    bias_vec = p_v[_K + 1, :]
    idx0 = p_v[_K + 2, :].astype(jnp.int32)
    inf = _full_f(jnp.inf)
    zero = _full_f(0.0)

    wb = [_bcast(w_row, k) for k in range(_K - 1)]

    sel_vec = jnp.where(iota == _full_i(0), idx0, _full_i(0))
    mask = iota == idx0
    cols = [plsc.load_gather(p_v, [iota, idx0])]

    for L in range(1, _K):
        score = bias_vec
        for j in range(L):
            score = score + cols[j] * wb[L - 1 - j]
        score = jnp.maximum(score, zero)
        score = jnp.where(score == zero, inf, score)
        score = jnp.where(mask, inf, score)
        m = jnp.min(score)
        cand = jnp.where(
            jnp.logical_and(jnp.logical_not(mask), score == _full_f(m)),
            iota,
            _full_i(_K),
        )
        nv = _full_i(jnp.min(cand))
        sel_vec = jnp.where(iota == _full_i(L), nv, sel_vec)
        mask = jnp.logical_or(mask, iota == nv)
        if L < _K - 1:
            cols.append(plsc.load_gather(p_v, [iota, nv]))

    o_v[...] = sel_vec
    pltpu.sync_copy(o_v, out_hbm)


def kernel(distance_matrixA, weight, bias, index_ls):
    packed = jnp.concatenate(
        [
            distance_matrixA,
            jnp.zeros((1, _K), jnp.float32).at[0, : weight.shape[1]].set(weight[0]),
            jnp.full((1, _K), bias[0], dtype=jnp.float32),
            jnp.full((1, _K), index_ls[0].astype(jnp.float32), dtype=jnp.float32),
        ],
        axis=0,
    )
    mesh = plsc.VectorSubcoreMesh(
        core_axis_name="c", subcore_axis_name="s", num_cores=1, num_subcores=1
    )
    f = pl.kernel(
        _sc_body,
        out_type=jax.ShapeDtypeStruct((_K,), jnp.int32),
        mesh=mesh,
        compiler_params=pltpu.CompilerParams(needs_layout_passes=False),
        scratch_types=[
            pltpu.VMEM((_K + 3, _K), jnp.float32),
            pltpu.VMEM((_K,), jnp.int32),
        ],
    )
    out_index = f(packed)
    return (out_index, weight)
